# baseline (speedup 1.0000x reference)
import functools

import jax
import jax.numpy as jnp
from jax import lax
from jax.experimental import pallas as pl
from jax.experimental.pallas import tpu as pltpu
from jax.experimental.pallas import tpu_sc as plsc

_BATCH = 4
_SEQ = 8192
_D = 64
_NC, _NS = 2, 16
_NW = _NC * _NS
_DB = 8
_NSH = 2
_SH = _SEQ // _NSH
_CH = 2048
_NCHB = _SH // _CH
_NTASK = _BATCH * _NCHB
_L = 16

_mesh = plsc.VectorSubcoreMesh(core_axis_name="c", subcore_axis_name="s")


@functools.partial(
    pl.kernel,
    mesh=_mesh,
    out_type=(
        jax.ShapeDtypeStruct((_BATCH, _D, _SEQ), jnp.float32),
        jax.ShapeDtypeStruct((_BATCH, _D, _SEQ), jnp.float32),
    ),
    scratch_types=[
        pltpu.VMEM((_DB, _SEQ), jnp.float32),
        [pltpu.VMEM((_CH,), jnp.int32) for _ in range(2)],
        [pltpu.VMEM((_DB, _CH), jnp.float32) for _ in range(2)],
        pltpu.SemaphoreType.DMA((2,)),
        pltpu.SemaphoreType.DMA((2,)),
    ],
    compiler_params=pltpu.CompilerParams(needs_layout_passes=False),
)
def _gather_t(pos_hbm, sint_hbm, cost_hbm, out_sin, out_cos,
              slab, pbufs, obufs, p_sem, w_sem):
    wid = lax.axis_index("s") * _NC + lax.axis_index("c")
    table = wid // (_NW // 2)
    dblk = (wid % (_NW // 2)) // _NSH
    shalf = wid % _NSH

    @pl.when(table == 0)
    def _():
        pltpu.sync_copy(sint_hbm.at[pl.ds(dblk * _DB, _DB)], slab)

    @pl.when(table == 1)
    def _():
        pltpu.sync_copy(cost_hbm.at[pl.ds(dblk * _DB, _DB)], slab)

    def start_pos(t):
        b, c = t // _NCHB, t % _NCHB
        src = pos_hbm.at[b, pl.ds(shalf * _SH + c * _CH, _CH)]
        return pltpu.async_copy(src, pbufs[t % 2], p_sem.at[t % 2])

    def _wb_dst(out, t):
        b, c = t // _NCHB, t % _NCHB
        return out.at[b, pl.ds(dblk * _DB, _DB),
                      pl.ds(shalf * _SH + c * _CH, _CH)]

    def start_wb(t):
        @pl.when(table == 0)
        def _():
            pltpu.async_copy(obufs[t % 2], _wb_dst(out_sin, t), w_sem.at[t % 2])

        @pl.when(table == 1)
        def _():
            pltpu.async_copy(obufs[t % 2], _wb_dst(out_cos, t), w_sem.at[t % 2])

        return pltpu.make_async_copy(obufs[t % 2], _wb_dst(out_sin, t),
                                     w_sem.at[t % 2])

    def fill(t):
        pb, ob = pbufs[t % 2], obufs[t % 2]
        rows = [jnp.full((_L,), d, jnp.int32) for d in range(_DB)]

        @plsc.parallel_loop(0, _CH, _L, unroll=8)
        def _(s):
            pvec = pb[pl.ds(s, _L)]
            for d in range(_DB):
                ob[d, pl.ds(s, _L)] = plsc.load_gather(slab, [rows[d], pvec])

    poss = {0: start_pos(0), 1: start_pos(1)}
    wbs = {}
    for t in range(_NTASK):
        poss[t].wait()
        if t >= 2:
            wbs[t - 2].wait()
        fill(t)
        wbs[t] = start_wb(t)
        if t + 2 < _NTASK:
            poss[t + 2] = start_pos(t + 2)
    for t in range(_NTASK - 2, _NTASK):
        wbs[t].wait()


def kernel(position, sin_values, cos_values):
    sin_t, cos_t = _gather_t(position, sin_values.T, cos_values.T)
    return (
        jnp.transpose(sin_t, (0, 2, 1)),
        jnp.transpose(cos_t, (0, 2, 1)),
    )


# speedup vs baseline: 1.0074x; 1.0074x over previous
import functools

import jax
import jax.numpy as jnp
from jax import lax
from jax.experimental import pallas as pl
from jax.experimental.pallas import tpu as pltpu
from jax.experimental.pallas import tpu_sc as plsc

_BATCH = 4
_SEQ = 8192
_D = 64
_NC, _NS = 2, 16
_NW = _NC * _NS
_DB = 8
_NSH = 2
_SH = _SEQ // _NSH
_CH = 2048
_NCHB = _SH // _CH
_NTASK = _BATCH * _NCHB
_L = 16

_mesh = plsc.VectorSubcoreMesh(core_axis_name="c", subcore_axis_name="s")


@functools.partial(
    pl.kernel,
    mesh=_mesh,
    out_type=(
        jax.ShapeDtypeStruct((_BATCH, _D, _SEQ), jnp.float32),
        jax.ShapeDtypeStruct((_BATCH, _D, _SEQ), jnp.float32),
    ),
    scratch_types=[
        pltpu.VMEM((_DB, _SEQ), jnp.float32),
        [pltpu.VMEM((_CH,), jnp.int32) for _ in range(2)],
        [pltpu.VMEM((_DB, _CH), jnp.float32) for _ in range(2)],
        pltpu.SemaphoreType.DMA((2,)),
        pltpu.SemaphoreType.DMA((2,)),
    ],
    compiler_params=pltpu.CompilerParams(needs_layout_passes=False),
)
def _gather_t(pos_hbm, sint_hbm, cost_hbm, out_sin, out_cos,
              slab, pbufs, obufs, p_sem, w_sem):
    wid = lax.axis_index("s") * _NC + lax.axis_index("c")
    table = wid // (_NW // 2)
    dblk = (wid % (_NW // 2)) // _NSH
    shalf = wid % _NSH

    @pl.when(table == 0)
    def _():
        pltpu.sync_copy(sint_hbm.at[pl.ds(dblk * _DB, _DB)], slab)

    @pl.when(table == 1)
    def _():
        pltpu.sync_copy(cost_hbm.at[pl.ds(dblk * _DB, _DB)], slab)

    def start_pos(t):
        b, c = t // _NCHB, t % _NCHB
        src = pos_hbm.at[b, pl.ds(shalf * _SH + c * _CH, _CH)]
        return pltpu.async_copy(src, pbufs[t % 2], p_sem.at[t % 2])

    def _wb_dst(out, t):
        b, c = t // _NCHB, t % _NCHB
        return out.at[b, pl.ds(dblk * _DB, _DB),
                      pl.ds(shalf * _SH + c * _CH, _CH)]

    def start_wb(t):
        @pl.when(table == 0)
        def _():
            pltpu.async_copy(obufs[t % 2], _wb_dst(out_sin, t), w_sem.at[t % 2])

        @pl.when(table == 1)
        def _():
            pltpu.async_copy(obufs[t % 2], _wb_dst(out_cos, t), w_sem.at[t % 2])

        return pltpu.make_async_copy(obufs[t % 2], _wb_dst(out_sin, t),
                                     w_sem.at[t % 2])

    def fill(t):
        pb, ob = pbufs[t % 2], obufs[t % 2]
        rows = [jnp.full((_L,), d, jnp.int32) for d in range(_DB)]

        @plsc.parallel_loop(0, _CH, _L, unroll=4)
        def _(s):
            pvec = pb[pl.ds(s, _L)]
            for d in range(_DB):
                ob[d, pl.ds(s, _L)] = plsc.load_gather(slab, [rows[d], pvec])

    poss = {0: start_pos(0), 1: start_pos(1)}
    wbs = {}
    for t in range(_NTASK):
        poss[t].wait()
        if t >= 2:
            wbs[t - 2].wait()
        fill(t)
        wbs[t] = start_wb(t)
        if t + 2 < _NTASK:
            poss[t + 2] = start_pos(t + 2)
    for t in range(_NTASK - 2, _NTASK):
        wbs[t].wait()


def kernel(position, sin_values, cos_values):
    sin_t, cos_t = _gather_t(position, sin_values.T, cos_values.T)
    return (
        jnp.transpose(sin_t, (0, 2, 1)),
        jnp.transpose(cos_t, (0, 2, 1)),
    )
